# Initial kernel scaffold; baseline (speedup 1.0000x reference)
import jax
import jax.numpy as jnp
from jax.experimental import pallas as pl


def kernel(features, edge_index, W1, b1, W2, b2):
    raise NotImplementedError("write your pallas kernel here")


# speedup vs baseline: 3.4168x; 3.4168x over previous
import functools

import jax
import jax.numpy as jnp
from jax import lax
from jax.experimental import pallas as pl
from jax.experimental.pallas import tpu as pltpu
from jax.experimental.pallas import tpu_sc as plsc

LANES = 16
BLK = 128
NC = 2
NS = 16
NW = NC * NS
IDX_CHUNK = 16


def _stage_row(dst_1d, src_2d, j):
    for c in range(BLK // LANES):
        dst_1d[pl.ds(c * LANES, LANES)] = src_2d[j, pl.ds(c * LANES, LANES)]


def _make_ones_agg_kernel(n_acc, num_blocks, d):
    rows_per_tile = n_acc // NS
    mesh = plsc.VectorSubcoreMesh(core_axis_name="c", subcore_axis_name="s")

    @functools.partial(
        pl.kernel, mesh=mesh,
        out_type=jax.ShapeDtypeStruct((NC, n_acc, d), jnp.float32),
        scratch_types=[
            pltpu.VMEM((IDX_CHUNK, BLK), jnp.int32),
            pltpu.VMEM((BLK,), jnp.int32),
            pltpu.VMEM((BLK, d), jnp.float32),
            pltpu.VMEM((BLK, d), jnp.float32),
            pltpu.VMEM_SHARED((n_acc, d), jnp.float32),
        ])
    def ones_agg_kernel(idx_hbm, ones_hbm, zeros_hbm, out_hbm,
                        idx_v, idx1, ones_v, zeros_v, acc):
        cid = lax.axis_index("c")
        sid = lax.axis_index("s")
        w = cid * NS + sid

        pltpu.sync_copy(ones_hbm, ones_v)
        pltpu.sync_copy(zeros_hbm, zeros_v)

        base = sid * rows_per_tile

        @pl.loop(0, rows_per_tile // BLK)
        def _(b):
            pltpu.sync_copy(zeros_v, acc.at[pl.ds(base + b * BLK, BLK)])

        plsc.subcore_barrier()

        @pl.loop(0, num_blocks // IDX_CHUNK)
        def _(cc):
            csl = pl.ds(cc * IDX_CHUNK, IDX_CHUNK)
            pltpu.sync_copy(idx_hbm.at[w].at[csl], idx_v)

            @pl.loop(0, IDX_CHUNK)
            def _(j):
                _stage_row(idx1, idx_v, j)
                pltpu.sync_copy(ones_v, acc.at[idx1], add=True)

        plsc.subcore_barrier()
        sl = pl.ds(base, rows_per_tile)
        pltpu.sync_copy(acc.at[sl], out_hbm.at[cid].at[sl])

    return ones_agg_kernel


def _make_agg_kernel(n_acc, num_blocks, d):
    rows_per_tile = n_acc // NS
    mesh = plsc.VectorSubcoreMesh(core_axis_name="c", subcore_axis_name="s")

    @functools.partial(
        pl.kernel, mesh=mesh,
        out_type=jax.ShapeDtypeStruct((NC, n_acc, d), jnp.float32),
        scratch_types=[
            pltpu.VMEM((IDX_CHUNK, BLK), jnp.int32),
            pltpu.VMEM((IDX_CHUNK, BLK), jnp.int32),
            pltpu.VMEM((BLK,), jnp.int32),
            pltpu.VMEM((BLK,), jnp.int32),
            pltpu.VMEM((BLK,), jnp.int32),
            pltpu.VMEM((BLK,), jnp.int32),
            pltpu.VMEM((BLK, d), jnp.float32),
            pltpu.VMEM((BLK, d), jnp.float32),
            pltpu.VMEM_SHARED((n_acc, d), jnp.float32),
            pltpu.SemaphoreType.DMA,
            pltpu.SemaphoreType.DMA,
        ])
    def agg_kernel(table_hbm, src_hbm, dst_hbm, out_hbm,
                   src_v, dst_v, s0, s1, d0, d1, buf0, buf1, acc, sem0, sem1):
        cid = lax.axis_index("c")
        sid = lax.axis_index("s")
        w = cid * NS + sid
        zero = jnp.zeros((LANES,), jnp.float32)

        @pl.loop(0, BLK)
        def _(r):
            for col in range(d // LANES):
                buf0[r, pl.ds(col * LANES, LANES)] = zero

        base = sid * rows_per_tile

        @pl.loop(0, rows_per_tile // BLK)
        def _(b):
            pltpu.sync_copy(buf0, acc.at[pl.ds(base + b * BLK, BLK)])

        plsc.subcore_barrier()

        @pl.loop(0, num_blocks // IDX_CHUNK)
        def _(cc):
            csl = pl.ds(cc * IDX_CHUNK, IDX_CHUNK)
            pltpu.sync_copy(src_hbm.at[w].at[csl], src_v)
            pltpu.sync_copy(dst_hbm.at[w].at[csl], dst_v)

            @pl.loop(0, IDX_CHUNK, step=2)
            def _(j):
                _stage_row(s0, src_v, j)
                _stage_row(s1, src_v, j + 1)
                g0 = pltpu.async_copy(table_hbm.at[s0], buf0, sem0)
                g1 = pltpu.async_copy(table_hbm.at[s1], buf1, sem1)
                _stage_row(d0, dst_v, j)
                _stage_row(d1, dst_v, j + 1)
                g0.wait()
                pltpu.sync_copy(buf0, acc.at[d0], add=True)
                g1.wait()
                pltpu.sync_copy(buf1, acc.at[d1], add=True)

        plsc.subcore_barrier()
        sl = pl.ds(base, rows_per_tile)
        pltpu.sync_copy(acc.at[sl], out_hbm.at[cid].at[sl])

    return agg_kernel


def _norm(tab):
    deg = jnp.sum(tab, axis=(0, 2)) * (1.0 / tab.shape[2])
    return lax.rsqrt(jnp.maximum(deg, 1.0))


def _tc_prescale_body(f_ref, dego_ref, o_ref):
    ns = _norm(dego_ref[...])
    n = f_ref.shape[0]
    o_ref[0:n, :] = f_ref[...] * ns[0:n, None]
    o_ref[n:, :] = jnp.zeros_like(o_ref[n:, :])


def _tc_layer1_body(p_ref, dego_ref, degi_ref, w_ref, b_ref, o_ref):
    ns = _norm(dego_ref[...])
    nd = _norm(degi_ref[...])
    agg = (p_ref[0] + p_ref[1]) * nd[:, None]
    h = jnp.dot(agg, w_ref[...], preferred_element_type=jnp.float32,
                precision=lax.Precision.HIGHEST) + b_ref[...]
    o_ref[...] = jnp.maximum(h, 0.0) * ns[:, None]


def _tc_layer2_body(q_ref, degi_ref, w_ref, b_ref, pool_ref, h_ref):
    nd = _norm(degi_ref[...])
    agg = (q_ref[0] + q_ref[1]) * nd[:, None]
    h2 = jnp.dot(agg, w_ref[...], preferred_element_type=jnp.float32,
                 precision=lax.Precision.HIGHEST) + b_ref[...]
    n = h_ref.shape[0]
    hn = h2[0:n, :]
    h_ref[...] = hn
    pool_ref[...] = jnp.sum(hn, axis=0, keepdims=True) * (1.0 / n)


def kernel(features, edge_index, W1, b1, W2, b2):
    n, d_in = features.shape
    e = edge_index.shape[1]
    d_hid = W1.shape[1]
    d_out = W2.shape[1]

    num_blocks = -(-e // (NW * BLK * IDX_CHUNK)) * IDX_CHUNK
    e_pad = NW * num_blocks * BLK
    rows_per_tile = -(-(n + 1) // (NS * BLK)) * BLK
    n_acc = NS * rows_per_tile

    pad = jnp.full((e_pad - e,), n, jnp.int32)
    src_p = jnp.concatenate([edge_index[0], pad]).reshape(NW, num_blocks, BLK)
    dst_p = jnp.concatenate([edge_index[1], pad]).reshape(NW, num_blocks, BLK)
    deg_w = 128
    ones_blk = jnp.full((BLK, deg_w), 1.0, jnp.float32)
    zeros_blk = jnp.zeros((BLK, deg_w), jnp.float32)
    mk_deg = _make_ones_agg_kernel(n_acc, num_blocks, deg_w)
    dego_p = mk_deg(src_p, ones_blk, zeros_blk)
    degi_p = mk_deg(dst_p, ones_blk, zeros_blk)

    xs = pl.pallas_call(
        _tc_prescale_body,
        out_shape=jax.ShapeDtypeStruct((n_acc, d_in), jnp.float32),
    )(features, dego_p)

    p1 = _make_agg_kernel(n_acc, num_blocks, d_in)(xs, src_p, dst_p)

    hs = pl.pallas_call(
        _tc_layer1_body,
        out_shape=jax.ShapeDtypeStruct((n_acc, d_hid), jnp.float32),
    )(p1, dego_p, degi_p, W1, b1.reshape(1, -1))

    p2 = _make_agg_kernel(n_acc, num_blocks, d_hid)(hs, src_p, dst_p)

    pool, h = pl.pallas_call(
        _tc_layer2_body,
        out_shape=[jax.ShapeDtypeStruct((1, d_out), jnp.float32),
                   jax.ShapeDtypeStruct((n, d_out), jnp.float32)],
    )(p2, degi_p, W2, b2.reshape(1, -1))
    return (pool, h)


# speedup vs baseline: 3.6498x; 1.0682x over previous
import functools

import jax
import jax.numpy as jnp
from jax import lax
from jax.experimental import pallas as pl
from jax.experimental.pallas import tpu as pltpu
from jax.experimental.pallas import tpu_sc as plsc

LANES = 16
BLK = 128
NC = 2
NS = 16
NW = NC * NS
IDX_CHUNK = 16


def _stage_row(dst_1d, src_2d, j):
    for c in range(BLK // LANES):
        dst_1d[pl.ds(c * LANES, LANES)] = src_2d[j, pl.ds(c * LANES, LANES)]


def _make_ones_agg_kernel(n_acc, num_blocks, d):
    rows_per_tile = n_acc // NS
    mesh = plsc.VectorSubcoreMesh(core_axis_name="c", subcore_axis_name="s")

    @functools.partial(
        pl.kernel, mesh=mesh,
        out_type=jax.ShapeDtypeStruct((NC, n_acc, d), jnp.float32),
        scratch_types=[
            pltpu.VMEM((IDX_CHUNK, BLK), jnp.int32),
            pltpu.VMEM((BLK,), jnp.int32),
            pltpu.VMEM((BLK, d), jnp.float32),
            pltpu.VMEM((BLK, d), jnp.float32),
            pltpu.VMEM_SHARED((n_acc, d), jnp.float32),
        ])
    def ones_agg_kernel(idx_hbm, ones_hbm, zeros_hbm, out_hbm,
                        idx_v, idx1, ones_v, zeros_v, acc):
        cid = lax.axis_index("c")
        sid = lax.axis_index("s")
        w = cid * NS + sid

        pltpu.sync_copy(ones_hbm, ones_v)
        pltpu.sync_copy(zeros_hbm, zeros_v)

        base = sid * rows_per_tile

        @pl.loop(0, rows_per_tile // BLK)
        def _(b):
            pltpu.sync_copy(zeros_v, acc.at[pl.ds(base + b * BLK, BLK)])

        plsc.subcore_barrier()

        @pl.loop(0, num_blocks // IDX_CHUNK)
        def _(cc):
            csl = pl.ds(cc * IDX_CHUNK, IDX_CHUNK)
            pltpu.sync_copy(idx_hbm.at[w].at[csl], idx_v)

            @pl.loop(0, IDX_CHUNK)
            def _(j):
                _stage_row(idx1, idx_v, j)
                pltpu.sync_copy(ones_v, acc.at[idx1], add=True)

        plsc.subcore_barrier()
        sl = pl.ds(base, rows_per_tile)
        pltpu.sync_copy(acc.at[sl], out_hbm.at[cid].at[sl])

    return ones_agg_kernel


def _make_agg_kernel(n_acc, num_blocks, d):
    rows_per_tile = n_acc // NS
    mesh = plsc.VectorSubcoreMesh(core_axis_name="c", subcore_axis_name="s")

    @functools.partial(
        pl.kernel, mesh=mesh,
        out_type=jax.ShapeDtypeStruct((NC, n_acc, d), jnp.float32),
        scratch_types=[
            pltpu.VMEM((IDX_CHUNK, BLK), jnp.int32),
            pltpu.VMEM((IDX_CHUNK, BLK), jnp.int32),
            pltpu.VMEM((BLK,), jnp.int32),
            pltpu.VMEM((BLK,), jnp.int32),
            pltpu.VMEM((BLK,), jnp.int32),
            pltpu.VMEM((BLK,), jnp.int32),
            pltpu.VMEM((BLK, d), jnp.float32),
            pltpu.VMEM((BLK, d), jnp.float32),
            pltpu.VMEM_SHARED((n_acc, d), jnp.float32),
            pltpu.SemaphoreType.DMA,
            pltpu.SemaphoreType.DMA,
            pltpu.SemaphoreType.DMA,
            pltpu.SemaphoreType.DMA,
        ])
    def agg_kernel(table_hbm, src_hbm, dst_hbm, out_hbm,
                   src_v, dst_v, s0, s1, d0, d1, buf0, buf1, acc,
                   sem0, sem1, sem_s0, sem_s1):
        cid = lax.axis_index("c")
        sid = lax.axis_index("s")
        w = cid * NS + sid
        zero = jnp.zeros((LANES,), jnp.float32)

        @pl.loop(0, BLK)
        def _(r):
            for col in range(d // LANES):
                buf0[r, pl.ds(col * LANES, LANES)] = zero

        base = sid * rows_per_tile

        @pl.loop(0, rows_per_tile // BLK)
        def _(b):
            pltpu.sync_copy(buf0, acc.at[pl.ds(base + b * BLK, BLK)])

        plsc.subcore_barrier()

        garb = pl.ds(n_acc - BLK, BLK)
        pltpu.async_copy(buf0, acc.at[garb], sem_s0)
        pltpu.async_copy(buf1, acc.at[garb], sem_s1)

        def _wait_scat(buf, sem):
            pltpu.make_async_copy(table_hbm.at[pl.ds(0, BLK)], buf, sem).wait()

        @pl.loop(0, num_blocks // IDX_CHUNK)
        def _(cc):
            csl = pl.ds(cc * IDX_CHUNK, IDX_CHUNK)
            pltpu.sync_copy(src_hbm.at[w].at[csl], src_v)
            pltpu.sync_copy(dst_hbm.at[w].at[csl], dst_v)

            @pl.loop(0, IDX_CHUNK, step=2)
            def _(j):
                _wait_scat(buf0, sem_s0)
                _stage_row(s0, src_v, j)
                _stage_row(d0, dst_v, j)
                g0 = pltpu.async_copy(table_hbm.at[s0], buf0, sem0)
                _wait_scat(buf1, sem_s1)
                _stage_row(s1, src_v, j + 1)
                _stage_row(d1, dst_v, j + 1)
                g1 = pltpu.async_copy(table_hbm.at[s1], buf1, sem1)
                g0.wait()
                pltpu.async_copy(buf0, acc.at[d0], sem_s0, add=True)
                g1.wait()
                pltpu.async_copy(buf1, acc.at[d1], sem_s1, add=True)

        _wait_scat(buf0, sem_s0)
        _wait_scat(buf1, sem_s1)
        plsc.subcore_barrier()
        sl = pl.ds(base, rows_per_tile)
        pltpu.sync_copy(acc.at[sl], out_hbm.at[cid].at[sl])

    return agg_kernel


def _norm(tab):
    deg = jnp.sum(tab, axis=(0, 2)) * (1.0 / tab.shape[2])
    return lax.rsqrt(jnp.maximum(deg, 1.0))


def _tc_prescale_body(f_ref, dego_ref, o_ref):
    ns = _norm(dego_ref[...])
    n = f_ref.shape[0]
    o_ref[0:n, :] = f_ref[...] * ns[0:n, None]
    o_ref[n:, :] = jnp.zeros_like(o_ref[n:, :])


def _tc_layer1_body(p_ref, dego_ref, degi_ref, w_ref, b_ref, o_ref):
    ns = _norm(dego_ref[...])
    nd = _norm(degi_ref[...])
    agg = (p_ref[0] + p_ref[1]) * nd[:, None]
    h = jnp.dot(agg, w_ref[...], preferred_element_type=jnp.float32,
                precision=lax.Precision.HIGHEST) + b_ref[...]
    o_ref[...] = jnp.maximum(h, 0.0) * ns[:, None]


def _tc_layer2_body(q_ref, degi_ref, w_ref, b_ref, pool_ref, h_ref):
    nd = _norm(degi_ref[...])
    agg = (q_ref[0] + q_ref[1]) * nd[:, None]
    h2 = jnp.dot(agg, w_ref[...], preferred_element_type=jnp.float32,
                 precision=lax.Precision.HIGHEST) + b_ref[...]
    n = h_ref.shape[0]
    hn = h2[0:n, :]
    h_ref[...] = hn
    pool_ref[...] = jnp.sum(hn, axis=0, keepdims=True) * (1.0 / n)


def kernel(features, edge_index, W1, b1, W2, b2):
    n, d_in = features.shape
    e = edge_index.shape[1]
    d_hid = W1.shape[1]
    d_out = W2.shape[1]

    num_blocks = -(-e // (NW * BLK * IDX_CHUNK)) * IDX_CHUNK
    e_pad = NW * num_blocks * BLK
    rows_per_tile = -(-(n + BLK) // (NS * BLK)) * BLK
    n_acc = NS * rows_per_tile

    pad = jnp.full((e_pad - e,), n, jnp.int32)
    src_p = jnp.concatenate([edge_index[0], pad]).reshape(NW, num_blocks, BLK)
    dst_p = jnp.concatenate([edge_index[1], pad]).reshape(NW, num_blocks, BLK)
    deg_w = 128
    ones_blk = jnp.full((BLK, deg_w), 1.0, jnp.float32)
    zeros_blk = jnp.zeros((BLK, deg_w), jnp.float32)
    mk_deg = _make_ones_agg_kernel(n_acc, num_blocks, deg_w)
    dego_p = mk_deg(src_p, ones_blk, zeros_blk)
    degi_p = mk_deg(dst_p, ones_blk, zeros_blk)

    xs = pl.pallas_call(
        _tc_prescale_body,
        out_shape=jax.ShapeDtypeStruct((n_acc, d_in), jnp.float32),
    )(features, dego_p)

    p1 = _make_agg_kernel(n_acc, num_blocks, d_in)(xs, src_p, dst_p)

    hs = pl.pallas_call(
        _tc_layer1_body,
        out_shape=jax.ShapeDtypeStruct((n_acc, d_hid), jnp.float32),
    )(p1, dego_p, degi_p, W1, b1.reshape(1, -1))

    p2 = _make_agg_kernel(n_acc, num_blocks, d_hid)(hs, src_p, dst_p)

    pool, h = pl.pallas_call(
        _tc_layer2_body,
        out_shape=[jax.ShapeDtypeStruct((1, d_out), jnp.float32),
                   jax.ShapeDtypeStruct((n, d_out), jnp.float32)],
    )(p2, degi_p, W2, b2.reshape(1, -1))
    return (pool, h)


# speedup vs baseline: 3.6503x; 1.0001x over previous
import functools

import jax
import jax.numpy as jnp
from jax import lax
from jax.experimental import pallas as pl
from jax.experimental.pallas import tpu as pltpu
from jax.experimental.pallas import tpu_sc as plsc

LANES = 16
BLK = 128
NC = 2
NS = 16
NW = NC * NS
IDX_CHUNK = 16


def _stage_row(dst_1d, src_2d, j):
    for c in range(BLK // LANES):
        dst_1d[pl.ds(c * LANES, LANES)] = src_2d[j, pl.ds(c * LANES, LANES)]


def _stage_half(dst_1d, src_2d, j, half):
    off = half * (BLK // 2)
    for c in range(BLK // (2 * LANES)):
        dst_1d[pl.ds(c * LANES, LANES)] = src_2d[j, pl.ds(off + c * LANES, LANES)]


def _make_ones_agg_kernel(n_acc, num_blocks, d):
    rows_per_tile = n_acc // NS
    mesh = plsc.VectorSubcoreMesh(core_axis_name="c", subcore_axis_name="s")

    @functools.partial(
        pl.kernel, mesh=mesh,
        out_type=jax.ShapeDtypeStruct((NC, n_acc, d), jnp.float32),
        scratch_types=[
            pltpu.VMEM((IDX_CHUNK, BLK), jnp.int32),
            pltpu.VMEM((BLK,), jnp.int32),
            pltpu.VMEM((BLK, d), jnp.float32),
            pltpu.VMEM((BLK, d), jnp.float32),
            pltpu.VMEM_SHARED((n_acc, d), jnp.float32),
        ])
    def ones_agg_kernel(idx_hbm, ones_hbm, zeros_hbm, out_hbm,
                        idx_v, idx1, ones_v, zeros_v, acc):
        cid = lax.axis_index("c")
        sid = lax.axis_index("s")
        w = cid * NS + sid

        pltpu.sync_copy(ones_hbm, ones_v)
        pltpu.sync_copy(zeros_hbm, zeros_v)

        base = sid * rows_per_tile

        @pl.loop(0, rows_per_tile // BLK)
        def _(b):
            pltpu.sync_copy(zeros_v, acc.at[pl.ds(base + b * BLK, BLK)])

        plsc.subcore_barrier()

        @pl.loop(0, num_blocks // IDX_CHUNK)
        def _(cc):
            csl = pl.ds(cc * IDX_CHUNK, IDX_CHUNK)
            pltpu.sync_copy(idx_hbm.at[w].at[csl], idx_v)

            @pl.loop(0, IDX_CHUNK)
            def _(j):
                _stage_row(idx1, idx_v, j)
                pltpu.sync_copy(ones_v, acc.at[idx1], add=True)

        plsc.subcore_barrier()
        sl = pl.ds(base, rows_per_tile)
        pltpu.sync_copy(acc.at[sl], out_hbm.at[cid].at[sl])

    return ones_agg_kernel


def _make_agg_kernel(n_acc, num_blocks, d):
    rows_per_tile = n_acc // NS
    mesh = plsc.VectorSubcoreMesh(core_axis_name="c", subcore_axis_name="s")

    @functools.partial(
        pl.kernel, mesh=mesh,
        out_type=jax.ShapeDtypeStruct((NC, n_acc, d), jnp.float32),
        scratch_types=[
            pltpu.VMEM((IDX_CHUNK, BLK), jnp.int32),
            pltpu.VMEM((IDX_CHUNK, BLK), jnp.int32),
            pltpu.VMEM((BLK // 2,), jnp.int32),
            pltpu.VMEM((BLK // 2,), jnp.int32),
            pltpu.VMEM((BLK // 2,), jnp.int32),
            pltpu.VMEM((BLK // 2,), jnp.int32),
            pltpu.VMEM((BLK,), jnp.int32),
            pltpu.VMEM((BLK,), jnp.int32),
            pltpu.VMEM((BLK, d), jnp.float32),
            pltpu.VMEM((BLK, d), jnp.float32),
            pltpu.VMEM_SHARED((n_acc, d), jnp.float32),
            pltpu.SemaphoreType.DMA,
            pltpu.SemaphoreType.DMA,
            pltpu.SemaphoreType.DMA,
            pltpu.SemaphoreType.DMA,
            pltpu.SemaphoreType.DMA,
            pltpu.SemaphoreType.DMA,
        ])
    def agg_kernel(table_hbm, src_hbm, dst_hbm, out_hbm,
                   src_v, dst_v, s0a, s0b, s1a, s1b, d0, d1, buf0, buf1, acc,
                   sem0a, sem0b, sem1a, sem1b, sem_s0, sem_s1):
        cid = lax.axis_index("c")
        sid = lax.axis_index("s")
        w = cid * NS + sid
        zero = jnp.zeros((LANES,), jnp.float32)

        @pl.loop(0, BLK)
        def _(r):
            for col in range(d // LANES):
                buf0[r, pl.ds(col * LANES, LANES)] = zero

        base = sid * rows_per_tile

        @pl.loop(0, rows_per_tile // BLK)
        def _(b):
            pltpu.sync_copy(buf0, acc.at[pl.ds(base + b * BLK, BLK)])

        plsc.subcore_barrier()

        garb = pl.ds(n_acc - BLK, BLK)
        pltpu.async_copy(buf0, acc.at[garb], sem_s0)
        pltpu.async_copy(buf1, acc.at[garb], sem_s1)

        def _wait_scat(buf, sem):
            pltpu.make_async_copy(table_hbm.at[pl.ds(0, BLK)], buf, sem).wait()

        @pl.loop(0, num_blocks // IDX_CHUNK)
        def _(cc):
            csl = pl.ds(cc * IDX_CHUNK, IDX_CHUNK)
            pltpu.sync_copy(src_hbm.at[w].at[csl], src_v)
            pltpu.sync_copy(dst_hbm.at[w].at[csl], dst_v)

            @pl.loop(0, IDX_CHUNK, step=2)
            def _(j):
                _wait_scat(buf0, sem_s0)
                _stage_half(s0a, src_v, j, 0)
                _stage_half(s0b, src_v, j, 1)
                _stage_row(d0, dst_v, j)
                g0a = pltpu.async_copy(table_hbm.at[s0a],
                                       buf0.at[pl.ds(0, BLK // 2)], sem0a)
                g0b = pltpu.async_copy(table_hbm.at[s0b],
                                       buf0.at[pl.ds(BLK // 2, BLK // 2)], sem0b)
                _wait_scat(buf1, sem_s1)
                _stage_half(s1a, src_v, j + 1, 0)
                _stage_half(s1b, src_v, j + 1, 1)
                _stage_row(d1, dst_v, j + 1)
                g1a = pltpu.async_copy(table_hbm.at[s1a],
                                       buf1.at[pl.ds(0, BLK // 2)], sem1a)
                g1b = pltpu.async_copy(table_hbm.at[s1b],
                                       buf1.at[pl.ds(BLK // 2, BLK // 2)], sem1b)
                g0a.wait()
                g0b.wait()
                pltpu.async_copy(buf0, acc.at[d0], sem_s0, add=True)
                g1a.wait()
                g1b.wait()
                pltpu.async_copy(buf1, acc.at[d1], sem_s1, add=True)

        _wait_scat(buf0, sem_s0)
        _wait_scat(buf1, sem_s1)
        plsc.subcore_barrier()
        sl = pl.ds(base, rows_per_tile)
        pltpu.sync_copy(acc.at[sl], out_hbm.at[cid].at[sl])

    return agg_kernel


def _norm(tab):
    deg = jnp.sum(tab, axis=(0, 2)) * (1.0 / tab.shape[2])
    return lax.rsqrt(jnp.maximum(deg, 1.0))


def _tc_prescale_body(f_ref, dego_ref, o_ref):
    ns = _norm(dego_ref[...])
    n = f_ref.shape[0]
    o_ref[0:n, :] = f_ref[...] * ns[0:n, None]
    o_ref[n:, :] = jnp.zeros_like(o_ref[n:, :])


def _tc_layer1_body(p_ref, dego_ref, degi_ref, w_ref, b_ref, o_ref):
    ns = _norm(dego_ref[...])
    nd = _norm(degi_ref[...])
    agg = (p_ref[0] + p_ref[1]) * nd[:, None]
    h = jnp.dot(agg, w_ref[...], preferred_element_type=jnp.float32,
                precision=lax.Precision.HIGHEST) + b_ref[...]
    o_ref[...] = jnp.maximum(h, 0.0) * ns[:, None]


def _tc_layer2_body(q_ref, degi_ref, w_ref, b_ref, pool_ref, h_ref):
    nd = _norm(degi_ref[...])
    agg = (q_ref[0] + q_ref[1]) * nd[:, None]
    h2 = jnp.dot(agg, w_ref[...], preferred_element_type=jnp.float32,
                 precision=lax.Precision.HIGHEST) + b_ref[...]
    n = h_ref.shape[0]
    hn = h2[0:n, :]
    h_ref[...] = hn
    pool_ref[...] = jnp.sum(hn, axis=0, keepdims=True) * (1.0 / n)


def kernel(features, edge_index, W1, b1, W2, b2):
    n, d_in = features.shape
    e = edge_index.shape[1]
    d_hid = W1.shape[1]
    d_out = W2.shape[1]

    num_blocks = -(-e // (NW * BLK * IDX_CHUNK)) * IDX_CHUNK
    e_pad = NW * num_blocks * BLK
    rows_per_tile = -(-(n + BLK) // (NS * BLK)) * BLK
    n_acc = NS * rows_per_tile

    pad = jnp.full((e_pad - e,), n, jnp.int32)
    src_p = jnp.concatenate([edge_index[0], pad]).reshape(NW, num_blocks, BLK)
    dst_p = jnp.concatenate([edge_index[1], pad]).reshape(NW, num_blocks, BLK)
    deg_w = 128
    ones_blk = jnp.full((BLK, deg_w), 1.0, jnp.float32)
    zeros_blk = jnp.zeros((BLK, deg_w), jnp.float32)
    mk_deg = _make_ones_agg_kernel(n_acc, num_blocks, deg_w)
    dego_p = mk_deg(src_p, ones_blk, zeros_blk)
    degi_p = mk_deg(dst_p, ones_blk, zeros_blk)

    xs = pl.pallas_call(
        _tc_prescale_body,
        out_shape=jax.ShapeDtypeStruct((n_acc, d_in), jnp.float32),
    )(features, dego_p)

    p1 = _make_agg_kernel(n_acc, num_blocks, d_in)(xs, src_p, dst_p)

    hs = pl.pallas_call(
        _tc_layer1_body,
        out_shape=jax.ShapeDtypeStruct((n_acc, d_hid), jnp.float32),
    )(p1, dego_p, degi_p, W1, b1.reshape(1, -1))

    p2 = _make_agg_kernel(n_acc, num_blocks, d_hid)(hs, src_p, dst_p)

    pool, h = pl.pallas_call(
        _tc_layer2_body,
        out_shape=[jax.ShapeDtypeStruct((1, d_out), jnp.float32),
                   jax.ShapeDtypeStruct((n, d_out), jnp.float32)],
    )(p2, degi_p, W2, b2.reshape(1, -1))
    return (pool, h)
